# Initial kernel scaffold; baseline (speedup 1.0000x reference)
import functools

import jax
import jax.numpy as jnp
from jax import lax
from jax.experimental import pallas as pl
from jax.experimental.pallas import tpu as pltpu
from jax.experimental.pallas import tpu_sc as plsc

N_USERS = 10000
N_NODES = 50000
N_EDGES = 800000
EMB_DIM = 64
BATCH = 1024

EW = 128
NWIN = N_EDGES // EW
NSUB = 16
NCORE = 2


def _spmm_sc(x, src2d, dst2d, ev2d, zeros, d, feature_split):
    n = N_NODES
    mesh = plsc.VectorSubcoreMesh(core_axis_name="c", subcore_axis_name="s")

    @functools.partial(
        pl.kernel,
        out_type=jax.ShapeDtypeStruct((2 * n, d), jnp.float32),
        mesh=mesh,
        scratch_types=[
            pltpu.VMEM((1, EW), jnp.int32),
            pltpu.VMEM((1, EW), jnp.int32),
            pltpu.VMEM((1, EW), jnp.float32),
            pltpu.VMEM((EW, d), jnp.float32),
            pltpu.VMEM_SHARED((n, d), jnp.float32),
        ],
    )
    def spmm(x_hbm, src_hbm, dst_hbm, ev_hbm, z_hbm, out_hbm,
             sidx, didx, evv, rows, side):
        c = lax.axis_index("c")
        s = lax.axis_index("s")
        stripe = n // NSUB
        pltpu.sync_copy(z_hbm.at[pl.ds(s * stripe, stripe)],
                        side.at[pl.ds(s * stripe, stripe)])
        plsc.subcore_barrier()

        if feature_split:
            base = 0
            nw = 390 + jnp.where(s < 10, 1, 0).astype(jnp.int32)
        else:
            base = c * (NWIN // 2)
            nw = 195 + jnp.where(s < 5, 1, 0).astype(jnp.int32)
        xoff = c * n

        @pl.loop(0, nw)
        def _(k):
            w = base + s + NSUB * k
            pltpu.sync_copy(src_hbm.at[w], sidx.at[0])
            pltpu.sync_copy(dst_hbm.at[w], didx.at[0])
            pltpu.sync_copy(ev_hbm.at[w], evv.at[0])
            if feature_split:
                for j in range(EW // 16):
                    sl = pl.ds(j * 16, 16)
                    sidx[0, sl] = sidx[0, sl] + xoff
            pltpu.sync_copy(x_hbm.at[sidx.at[0]], rows)
            @pl.loop(0, EW)
            def _(w2):
                sv = evv[0, w2]
                for j in range(d // 16):
                    sl = pl.ds(j * 16, 16)
                    rows[w2, sl] = rows[w2, sl] * sv
            pltpu.sync_copy(rows, side.at[didx.at[0]], add=True)

        plsc.subcore_barrier()
        pltpu.sync_copy(side.at[pl.ds(s * stripe, stripe)],
                        out_hbm.at[pl.ds(c * n + s * stripe, stripe)])

    return spmm(x, src2d, dst2d, ev2d, zeros)


def _dense_tc(x, side2, Wg, bg, Wb, bb, concat_mode, pad_to):
    n, d_in = x.shape
    d_out = Wg.shape[1]
    R = 2000
    nblk = n // R

    def body(x_ref, s0_ref, s1_ref, wg_ref, bg_ref, wb_ref, bb_ref, o_ref):
        xb = x_ref[...]
        if concat_mode:
            side = jnp.concatenate([s0_ref[...], s1_ref[...]], axis=1)
        else:
            side = s0_ref[...] + s1_ref[...]
        a = jnp.dot(xb + side, wg_ref[...],
                    preferred_element_type=jnp.float32) + bg_ref[...]
        a = jnp.where(a >= 0, a, 0.01 * a)
        b = jnp.dot(xb * side, wb_ref[...],
                    preferred_element_type=jnp.float32) + bb_ref[...]
        b = jnp.where(b >= 0, b, 0.01 * b)
        y = a + b
        nrm = jnp.sqrt(jnp.sum(y * y, axis=1, keepdims=True))
        y = y / jnp.maximum(nrm, 1e-12)
        if pad_to > d_out:
            y = jnp.concatenate(
                [y, jnp.zeros((y.shape[0], pad_to - d_out), jnp.float32)],
                axis=1)
        o_ref[...] = y

    d_side = side2.shape[1]
    return pl.pallas_call(
        body,
        grid=(nblk,),
        in_specs=[
            pl.BlockSpec((R, d_in), lambda i: (i, 0)),
            pl.BlockSpec((R, d_side), lambda i: (i, 0)),
            pl.BlockSpec((R, d_side), lambda i: (i + nblk, 0)),
            pl.BlockSpec((d_in, d_out), lambda i: (0, 0)),
            pl.BlockSpec((1, d_out), lambda i: (0, 0)),
            pl.BlockSpec((d_in, d_out), lambda i: (0, 0)),
            pl.BlockSpec((1, d_out), lambda i: (0, 0)),
        ],
        out_specs=pl.BlockSpec((R, pad_to), lambda i: (i, 0)),
        out_shape=jax.ShapeDtypeStruct((n, pad_to), jnp.float32),
    )(x, side2, side2, Wg, bg, Wb, bb)


def _gather_sc(x0, x1, x2, x3, users, items):
    per_w = BATCH // (NCORE * NSUB)
    mesh = plsc.VectorSubcoreMesh(core_axis_name="c", subcore_axis_name="s")
    f32 = jnp.float32
    out_types = tuple(
        jax.ShapeDtypeStruct((BATCH, dd), f32) for dd in (64, 32, 16, 16)
    ) * 2

    @functools.partial(
        pl.kernel,
        out_type=out_types,
        mesh=mesh,
        scratch_types=[
            pltpu.VMEM((1, per_w), jnp.int32),
            pltpu.VMEM((per_w, 64), f32),
            pltpu.VMEM((per_w, 32), f32),
            pltpu.VMEM((per_w, 16), f32),
            pltpu.VMEM((per_w, 16), f32),
        ],
    )
    def gat(x0_hbm, x1_hbm, x2_hbm, x3_hbm, u_hbm, i_hbm,
            u0, u1, u2, u3, i0, i1, i2, i3,
            idx, r0, r1, r2, r3):
        c = lax.axis_index("c")
        s = lax.axis_index("s")
        wid = s * NCORE + c
        base = wid * per_w

        def do(ind_hbm, off, o0, o1, o2, o3):
            pltpu.sync_copy(ind_hbm.at[pl.ds(base, per_w)], idx.at[0])
            if off:
                for j in range(per_w // 16):
                    sl = pl.ds(j * 16, 16)
                    idx[0, sl] = idx[0, sl] + off
            pltpu.sync_copy(x0_hbm.at[idx.at[0]], r0)
            pltpu.sync_copy(r0, o0.at[pl.ds(base, per_w)])
            pltpu.sync_copy(x1_hbm.at[idx.at[0]], r1)
            pltpu.sync_copy(r1, o1.at[pl.ds(base, per_w)])
            pltpu.sync_copy(x2_hbm.at[idx.at[0]], r2)
            pltpu.sync_copy(r2, o2.at[pl.ds(base, per_w)])
            pltpu.sync_copy(x3_hbm.at[idx.at[0]], r3)
            pltpu.sync_copy(r3, o3.at[pl.ds(base, per_w)])

        do(u_hbm, 0, u0, u1, u2, u3)
        do(i_hbm, N_USERS, i0, i1, i2, i3)

    return gat(x0, x1, x2, x3, users, items)


def _dot_tc(parts):
    u0, u1, u2, u3, i0, i1, i2, i3 = parts

    def body(u0r, u1r, u2r, u3r, i0r, i1r, i2r, i3r, o_ref):
        acc = jnp.sum(u0r[...] * i0r[...], axis=1, keepdims=True)
        acc += jnp.sum(u1r[...] * i1r[...], axis=1, keepdims=True)
        acc += jnp.sum(u2r[...] * i2r[...], axis=1, keepdims=True)
        acc += jnp.sum(u3r[...] * i3r[...], axis=1, keepdims=True)
        o_ref[...] = acc

    out = pl.pallas_call(
        body,
        out_shape=jax.ShapeDtypeStruct((BATCH, 1), jnp.float32),
    )(u0, u1, u2, u3, i0, i1, i2, i3)
    return out.reshape(BATCH)


def kernel(edge_vals, user_embed, entity_embed,
           W_gc_0, b_gc_0, W_bi_0, b_bi_0,
           W_gc_1, b_gc_1, W_bi_1, b_bi_1,
           W_gc_2, b_gc_2, W_bi_2, b_bi_2,
           edge_index, users, pos_items):
    f32 = jnp.float32
    xcat = jnp.concatenate([user_embed, entity_embed], axis=0)
    dst2d = edge_index[0].reshape(NWIN, EW)
    src2d = edge_index[1].reshape(NWIN, EW)
    ev2d = edge_vals.reshape(NWIN, EW)
    z32 = jnp.zeros((N_NODES, 32), f32)
    z16 = jnp.zeros((N_NODES, 16), f32)

    xs0 = jnp.concatenate([xcat[:, :32], xcat[:, 32:]], axis=0)
    side0 = _spmm_sc(xs0, src2d, dst2d, ev2d, z32, 32, feature_split=True)
    x1 = _dense_tc(xcat, side0, W_gc_0, b_gc_0, W_bi_0, b_bi_0,
                   concat_mode=True, pad_to=32)

    side1 = _spmm_sc(x1, src2d, dst2d, ev2d, z32, 32, feature_split=False)
    x2 = _dense_tc(x1, side1, W_gc_1, b_gc_1, W_bi_1, b_bi_1,
                   concat_mode=False, pad_to=16)

    side2 = _spmm_sc(x2, src2d, dst2d, ev2d, z16, 16, feature_split=False)
    x3 = _dense_tc(x2, side2, W_gc_2, b_gc_2, W_bi_2, b_bi_2,
                   concat_mode=False, pad_to=16)

    parts = _gather_sc(xcat, x1, x2, x3, users, pos_items)
    return _dot_tc(parts)


# speedup vs baseline: 4.8816x; 4.8816x over previous
import functools

import jax
import jax.numpy as jnp
from jax import lax
from jax.experimental import pallas as pl
from jax.experimental.pallas import tpu as pltpu
from jax.experimental.pallas import tpu_sc as plsc

N_USERS = 10000
N_NODES = 50000
N_EDGES = 800000
EMB_DIM = 64
BATCH = 1024

EW = 128
NWIN = N_EDGES // EW
NSUB = 16
NCORE = 2


STRIPE = 3128
STRIPE_LAST = N_NODES - 15 * STRIPE


def _spmm_sc(x, src1d, dst1d, ev1d, zeros, d, feature_split):
    n = N_NODES
    mesh = plsc.VectorSubcoreMesh(core_axis_name="c", subcore_axis_name="s")

    @functools.partial(
        pl.kernel,
        out_type=jax.ShapeDtypeStruct((2 * n, d), jnp.float32),
        mesh=mesh,
        scratch_types=[
            pltpu.VMEM((1, EW), jnp.int32),
            pltpu.VMEM((1, EW), jnp.int32),
            pltpu.VMEM((1, EW), jnp.float32),
            pltpu.VMEM((EW, d), jnp.float32),
            pltpu.VMEM_SHARED((n, d), jnp.float32),
        ],
        compiler_params=pltpu.CompilerParams(use_tc_tiling_on_sc=False),
    )
    def spmm(x_hbm, src_hbm, dst_hbm, ev_hbm, z_hbm, out_hbm,
             sidx, didx, evv, rows, side):
        c = lax.axis_index("c")
        s = lax.axis_index("s")

        @pl.when(s < 15)
        def _():
            pltpu.sync_copy(z_hbm.at[pl.ds(s * STRIPE, STRIPE)],
                            side.at[pl.ds(s * STRIPE, STRIPE)])

        @pl.when(s == 15)
        def _():
            pltpu.sync_copy(z_hbm.at[pl.ds(15 * STRIPE, STRIPE_LAST)],
                            side.at[pl.ds(15 * STRIPE, STRIPE_LAST)])

        plsc.subcore_barrier()

        if feature_split:
            base = 0
            nw = 390 + jnp.where(s < 10, 1, 0).astype(jnp.int32)
        else:
            base = c * (NWIN // 2)
            nw = 195 + jnp.where(s < 5, 1, 0).astype(jnp.int32)
        xoff = c * n

        @pl.loop(0, nw)
        def _(k):
            w = base + s + NSUB * k
            eoff = w * EW
            pltpu.sync_copy(src_hbm.at[pl.ds(eoff, EW)], sidx.at[0])
            pltpu.sync_copy(dst_hbm.at[pl.ds(eoff, EW)], didx.at[0])
            pltpu.sync_copy(ev_hbm.at[pl.ds(eoff, EW)], evv.at[0])
            if feature_split:
                for j in range(EW // 16):
                    sl = pl.ds(j * 16, 16)
                    sidx[0, sl] = sidx[0, sl] + xoff
            pltpu.sync_copy(x_hbm.at[sidx.at[0]], rows)
            @pl.loop(0, EW // 16)
            def _(cc):
                ev16 = evv[0, pl.ds(cc * 16, 16)]
                for l in range(16):
                    sv = ev16[l]
                    w2 = cc * 16 + l
                    for j in range(d // 16):
                        sl = pl.ds(j * 16, 16)
                        rows[w2, sl] = rows[w2, sl] * sv
            pltpu.sync_copy(rows, side.at[didx.at[0]], add=True)

        plsc.subcore_barrier()

        @pl.when(s < 15)
        def _():
            pltpu.sync_copy(side.at[pl.ds(s * STRIPE, STRIPE)],
                            out_hbm.at[pl.ds(c * n + s * STRIPE, STRIPE)])

        @pl.when(s == 15)
        def _():
            pltpu.sync_copy(side.at[pl.ds(15 * STRIPE, STRIPE_LAST)],
                            out_hbm.at[pl.ds(c * n + 15 * STRIPE, STRIPE_LAST)])

    return spmm(x, src1d, dst1d, ev1d, zeros)


def _dense_tc(x, side2, Wg, bg, Wb, bb, concat_mode, pad_to):
    n, d_in = x.shape
    d_out = Wg.shape[1]
    R = 2000
    nblk = n // R

    def body(x_ref, s0_ref, s1_ref, wg_ref, bg_ref, wb_ref, bb_ref, o_ref):
        xb = x_ref[...]
        if concat_mode:
            side = jnp.concatenate([s0_ref[...], s1_ref[...]], axis=1)
        else:
            side = s0_ref[...] + s1_ref[...]
        a = jnp.dot(xb + side, wg_ref[...],
                    preferred_element_type=jnp.float32) + bg_ref[...]
        a = jnp.where(a >= 0, a, 0.01 * a)
        b = jnp.dot(xb * side, wb_ref[...],
                    preferred_element_type=jnp.float32) + bb_ref[...]
        b = jnp.where(b >= 0, b, 0.01 * b)
        y = a + b
        nrm = jnp.sqrt(jnp.sum(y * y, axis=1, keepdims=True))
        y = y / jnp.maximum(nrm, 1e-12)
        if pad_to > d_out:
            y = jnp.concatenate(
                [y, jnp.zeros((y.shape[0], pad_to - d_out), jnp.float32)],
                axis=1)
        o_ref[...] = y

    d_side = side2.shape[1]
    return pl.pallas_call(
        body,
        grid=(nblk,),
        in_specs=[
            pl.BlockSpec((R, d_in), lambda i: (i, 0)),
            pl.BlockSpec((R, d_side), lambda i: (i, 0)),
            pl.BlockSpec((R, d_side), lambda i: (i + nblk, 0)),
            pl.BlockSpec((d_in, d_out), lambda i: (0, 0)),
            pl.BlockSpec((1, d_out), lambda i: (0, 0)),
            pl.BlockSpec((d_in, d_out), lambda i: (0, 0)),
            pl.BlockSpec((1, d_out), lambda i: (0, 0)),
        ],
        out_specs=pl.BlockSpec((R, pad_to), lambda i: (i, 0)),
        out_shape=jax.ShapeDtypeStruct((n, pad_to), jnp.float32),
    )(x, side2, side2, Wg, bg, Wb, bb)


def _gather_sc(x0, x1, x2, x3, users, items):
    per_w = BATCH // (NCORE * NSUB)
    mesh = plsc.VectorSubcoreMesh(core_axis_name="c", subcore_axis_name="s")
    f32 = jnp.float32
    out_types = tuple(
        jax.ShapeDtypeStruct((BATCH, dd), f32) for dd in (64, 32, 16, 16)
    ) * 2

    @functools.partial(
        pl.kernel,
        out_type=out_types,
        mesh=mesh,
        scratch_types=[
            pltpu.VMEM((1, per_w), jnp.int32),
            pltpu.VMEM((per_w, 64), f32),
            pltpu.VMEM((per_w, 32), f32),
            pltpu.VMEM((per_w, 16), f32),
            pltpu.VMEM((per_w, 16), f32),
        ],
        compiler_params=pltpu.CompilerParams(use_tc_tiling_on_sc=False),
    )
    def gat(x0_hbm, x1_hbm, x2_hbm, x3_hbm, u_hbm, i_hbm,
            u0, u1, u2, u3, i0, i1, i2, i3,
            idx, r0, r1, r2, r3):
        c = lax.axis_index("c")
        s = lax.axis_index("s")
        wid = s * NCORE + c
        base = wid * per_w

        def do(ind_hbm, off, o0, o1, o2, o3):
            pltpu.sync_copy(ind_hbm.at[pl.ds(base, per_w)], idx.at[0])
            if off:
                for j in range(per_w // 16):
                    sl = pl.ds(j * 16, 16)
                    idx[0, sl] = idx[0, sl] + off
            pltpu.sync_copy(x0_hbm.at[idx.at[0]], r0)
            pltpu.sync_copy(r0, o0.at[pl.ds(base, per_w)])
            pltpu.sync_copy(x1_hbm.at[idx.at[0]], r1)
            pltpu.sync_copy(r1, o1.at[pl.ds(base, per_w)])
            pltpu.sync_copy(x2_hbm.at[idx.at[0]], r2)
            pltpu.sync_copy(r2, o2.at[pl.ds(base, per_w)])
            pltpu.sync_copy(x3_hbm.at[idx.at[0]], r3)
            pltpu.sync_copy(r3, o3.at[pl.ds(base, per_w)])

        do(u_hbm, 0, u0, u1, u2, u3)
        do(i_hbm, N_USERS, i0, i1, i2, i3)

    return gat(x0, x1, x2, x3, users, items)


def _dot_tc(parts):
    u0, u1, u2, u3, i0, i1, i2, i3 = parts

    def body(u0r, u1r, u2r, u3r, i0r, i1r, i2r, i3r, o_ref):
        acc = jnp.sum(u0r[...] * i0r[...], axis=1, keepdims=True)
        acc += jnp.sum(u1r[...] * i1r[...], axis=1, keepdims=True)
        acc += jnp.sum(u2r[...] * i2r[...], axis=1, keepdims=True)
        acc += jnp.sum(u3r[...] * i3r[...], axis=1, keepdims=True)
        o_ref[...] = acc

    out = pl.pallas_call(
        body,
        out_shape=jax.ShapeDtypeStruct((BATCH, 1), jnp.float32),
    )(u0, u1, u2, u3, i0, i1, i2, i3)
    return out.reshape(BATCH)


def kernel(edge_vals, user_embed, entity_embed,
           W_gc_0, b_gc_0, W_bi_0, b_bi_0,
           W_gc_1, b_gc_1, W_bi_1, b_bi_1,
           W_gc_2, b_gc_2, W_bi_2, b_bi_2,
           edge_index, users, pos_items):
    f32 = jnp.float32
    xcat = jnp.concatenate([user_embed, entity_embed], axis=0)
    dst1d = edge_index[0]
    src1d = edge_index[1]
    ev1d = edge_vals
    z32 = jnp.zeros((N_NODES, 32), f32)
    z16 = jnp.zeros((N_NODES, 16), f32)

    xs0 = jnp.concatenate([xcat[:, :32], xcat[:, 32:]], axis=0)
    side0 = _spmm_sc(xs0, src1d, dst1d, ev1d, z32, 32, feature_split=True)
    x1 = _dense_tc(xcat, side0, W_gc_0, b_gc_0, W_bi_0, b_bi_0,
                   concat_mode=True, pad_to=32)

    side1 = _spmm_sc(x1, src1d, dst1d, ev1d, z32, 32, feature_split=False)
    x2 = _dense_tc(x1, side1, W_gc_1, b_gc_1, W_bi_1, b_bi_1,
                   concat_mode=False, pad_to=16)

    side2 = _spmm_sc(x2, src1d, dst1d, ev1d, z16, 16, feature_split=False)
    x3 = _dense_tc(x2, side2, W_gc_2, b_gc_2, W_bi_2, b_bi_2,
                   concat_mode=False, pad_to=16)

    parts = _gather_sc(xcat, x1, x2, x3, users, pos_items)
    return _dot_tc(parts)


# speedup vs baseline: 10.2621x; 2.1022x over previous
import functools

import jax
import jax.numpy as jnp
from jax import lax
from jax.experimental import pallas as pl
from jax.experimental.pallas import tpu as pltpu
from jax.experimental.pallas import tpu_sc as plsc

N_USERS = 10000
N_NODES = 50000
N_EDGES = 800000
EMB_DIM = 64
BATCH = 1024

EW = 128
NSUB = 16
NCORE = 2
NWIN = 6336
E_PAD = NWIN * EW


STRIPE = 3128
STRIPE_LAST = N_NODES - 15 * STRIPE


def _spmm_sc(x, packed, zeros, d, feature_split, G):
    n = N_NODES
    mesh = plsc.VectorSubcoreMesh(core_axis_name="c", subcore_axis_name="s")

    if feature_split:
        wps = NWIN // NSUB
    else:
        wps = NWIN // NCORE // NSUB
    npairs = wps // (2 * G)
    assert npairs * 2 * G == wps

    @functools.partial(
        pl.kernel,
        out_type=jax.ShapeDtypeStruct((2 * n, d), jnp.float32),
        mesh=mesh,
        scratch_types=[
            pltpu.VMEM((G, 3, EW), jnp.int32),
            pltpu.VMEM((G * EW, d), jnp.float32),
            pltpu.VMEM((G, 3, EW), jnp.int32),
            pltpu.VMEM((G * EW, d), jnp.float32),
            pltpu.VMEM_SHARED((n, d), jnp.float32),
            pltpu.SemaphoreType.DMA,
            pltpu.SemaphoreType.DMA,
            pltpu.SemaphoreType.DMA,
            pltpu.SemaphoreType.DMA,
        ],
        compiler_params=pltpu.CompilerParams(use_tc_tiling_on_sc=False,
                                             needs_layout_passes=False),
    )
    def spmm(x_hbm, pk_hbm, z_hbm, out_hbm,
             pkA, rowsA, pkB, rowsB,
             side, gsemA, ssemA, gsemB, ssemB):
        c = lax.axis_index("c")
        s = lax.axis_index("s")

        @pl.when(s < 15)
        def _():
            pltpu.sync_copy(z_hbm.at[pl.ds(s * STRIPE, STRIPE)],
                            side.at[pl.ds(s * STRIPE, STRIPE)])

        @pl.when(s == 15)
        def _():
            pltpu.sync_copy(z_hbm.at[pl.ds(15 * STRIPE, STRIPE_LAST)],
                            side.at[pl.ds(15 * STRIPE, STRIPE_LAST)])

        plsc.subcore_barrier()

        if feature_split:
            base_w = s * wps
        else:
            base_w = c * (NWIN // NCORE) + s * wps
        xoff = c * n

        def do_load(S, gi):
            pk, rows, gsem, _ = S
            w0 = base_w + gi * G
            pltpu.sync_copy(pk_hbm.at[pl.ds(w0, G)], pk)
            if feature_split:
                for j in range(G):
                    for u in range(EW // 16):
                        sl = pl.ds(u * 16, 16)
                        pk[j, 0, sl] = pk[j, 0, sl] + xoff
            for j in range(G):
                pltpu.async_copy(x_hbm.at[pk.at[j, 0]],
                                 rows.at[pl.ds(j * EW, EW)], gsem)

        def drain_gathers(S):
            pk, rows, gsem, _ = S
            for j in range(G):
                pltpu.make_async_copy(x_hbm.at[pk.at[j, 0]],
                                      rows.at[pl.ds(j * EW, EW)], gsem).wait()

        def do_scale(S):
            pk, rows = S[0], S[1]

            @pl.loop(0, G)
            def _(j):
                @pl.loop(0, EW // 16)
                def _(cc):
                    ev16 = plsc.bitcast(pk[j, 2, pl.ds(cc * 16, 16)],
                                        jnp.float32)
                    r0 = j * EW + cc * 16
                    for l in range(16):
                        sv = ev16[l]
                        for jj in range(d // 16):
                            sl = pl.ds(jj * 16, 16)
                            rows[r0 + l, sl] = rows[r0 + l, sl] * sv

        def do_fire(S):
            pk, rows, _, ssem = S
            for j in range(G):
                pltpu.async_copy(rows.at[pl.ds(j * EW, EW)],
                                 side.at[pk.at[j, 1]], ssem, add=True)

        def drain_scatters(S):
            pk, rows, _, ssem = S
            for j in range(G):
                pltpu.make_async_copy(rows.at[pl.ds(j * EW, EW)],
                                      side.at[pk.at[j, 1]], ssem).wait()

        A = (pkA, rowsA, gsemA, ssemA)
        B = (pkB, rowsB, gsemB, ssemB)
        do_load(A, 0)

        @pl.loop(0, npairs)
        def _(p):
            drain_gathers(A)
            do_scale(A)

            @pl.when(p > 0)
            def _():
                drain_scatters(B)

            do_fire(A)
            do_load(B, 2 * p + 1)
            drain_gathers(B)
            do_scale(B)
            drain_scatters(A)
            do_fire(B)

            @pl.when(p < npairs - 1)
            def _():
                do_load(A, 2 * p + 2)

        drain_scatters(B)
        plsc.subcore_barrier()

        @pl.when(s < 15)
        def _():
            pltpu.sync_copy(side.at[pl.ds(s * STRIPE, STRIPE)],
                            out_hbm.at[pl.ds(c * n + s * STRIPE, STRIPE)])

        @pl.when(s == 15)
        def _():
            pltpu.sync_copy(side.at[pl.ds(15 * STRIPE, STRIPE_LAST)],
                            out_hbm.at[pl.ds(c * n + 15 * STRIPE, STRIPE_LAST)])

    return spmm(x, packed, zeros)


def _dense_tc(x, side2, Wg, bg, Wb, bb, concat_mode, pad_to):
    n, d_in = x.shape
    d_out = Wg.shape[1]
    R = 2000
    nblk = n // R

    def body(x_ref, s0_ref, s1_ref, wg_ref, bg_ref, wb_ref, bb_ref, o_ref):
        xb = x_ref[...]
        if concat_mode:
            side = jnp.concatenate([s0_ref[...], s1_ref[...]], axis=1)
        else:
            side = s0_ref[...] + s1_ref[...]
        a = jnp.dot(xb + side, wg_ref[...],
                    preferred_element_type=jnp.float32) + bg_ref[...]
        a = jnp.where(a >= 0, a, 0.01 * a)
        b = jnp.dot(xb * side, wb_ref[...],
                    preferred_element_type=jnp.float32) + bb_ref[...]
        b = jnp.where(b >= 0, b, 0.01 * b)
        y = a + b
        nrm = jnp.sqrt(jnp.sum(y * y, axis=1, keepdims=True))
        y = y / jnp.maximum(nrm, 1e-12)
        if pad_to > d_out:
            y = jnp.concatenate(
                [y, jnp.zeros((y.shape[0], pad_to - d_out), jnp.float32)],
                axis=1)
        o_ref[...] = y

    d_side = side2.shape[1]
    return pl.pallas_call(
        body,
        grid=(nblk,),
        in_specs=[
            pl.BlockSpec((R, d_in), lambda i: (i, 0)),
            pl.BlockSpec((R, d_side), lambda i: (i, 0)),
            pl.BlockSpec((R, d_side), lambda i: (i + nblk, 0)),
            pl.BlockSpec((d_in, d_out), lambda i: (0, 0)),
            pl.BlockSpec((1, d_out), lambda i: (0, 0)),
            pl.BlockSpec((d_in, d_out), lambda i: (0, 0)),
            pl.BlockSpec((1, d_out), lambda i: (0, 0)),
        ],
        out_specs=pl.BlockSpec((R, pad_to), lambda i: (i, 0)),
        out_shape=jax.ShapeDtypeStruct((n, pad_to), jnp.float32),
    )(x, side2, side2, Wg, bg, Wb, bb)


def _gather_sc(x0, x1, x2, x3, users, items):
    per_w = BATCH // (NCORE * NSUB)
    mesh = plsc.VectorSubcoreMesh(core_axis_name="c", subcore_axis_name="s")
    f32 = jnp.float32
    out_types = tuple(
        jax.ShapeDtypeStruct((BATCH, dd), f32) for dd in (64, 32, 16, 16)
    ) * 2

    @functools.partial(
        pl.kernel,
        out_type=out_types,
        mesh=mesh,
        scratch_types=[
            pltpu.VMEM((1, per_w), jnp.int32),
            pltpu.VMEM((per_w, 64), f32),
            pltpu.VMEM((per_w, 32), f32),
            pltpu.VMEM((per_w, 16), f32),
            pltpu.VMEM((per_w, 16), f32),
        ],
        compiler_params=pltpu.CompilerParams(use_tc_tiling_on_sc=False),
    )
    def gat(x0_hbm, x1_hbm, x2_hbm, x3_hbm, u_hbm, i_hbm,
            u0, u1, u2, u3, i0, i1, i2, i3,
            idx, r0, r1, r2, r3):
        c = lax.axis_index("c")
        s = lax.axis_index("s")
        wid = s * NCORE + c
        base = wid * per_w

        def do(ind_hbm, off, o0, o1, o2, o3):
            pltpu.sync_copy(ind_hbm.at[pl.ds(base, per_w)], idx.at[0])
            if off:
                for j in range(per_w // 16):
                    sl = pl.ds(j * 16, 16)
                    idx[0, sl] = idx[0, sl] + off
            pltpu.sync_copy(x0_hbm.at[idx.at[0]], r0)
            pltpu.sync_copy(r0, o0.at[pl.ds(base, per_w)])
            pltpu.sync_copy(x1_hbm.at[idx.at[0]], r1)
            pltpu.sync_copy(r1, o1.at[pl.ds(base, per_w)])
            pltpu.sync_copy(x2_hbm.at[idx.at[0]], r2)
            pltpu.sync_copy(r2, o2.at[pl.ds(base, per_w)])
            pltpu.sync_copy(x3_hbm.at[idx.at[0]], r3)
            pltpu.sync_copy(r3, o3.at[pl.ds(base, per_w)])

        do(u_hbm, 0, u0, u1, u2, u3)
        do(i_hbm, N_USERS, i0, i1, i2, i3)

    return gat(x0, x1, x2, x3, users, items)


def _dot_tc(parts):
    u0, u1, u2, u3, i0, i1, i2, i3 = parts

    def body(u0r, u1r, u2r, u3r, i0r, i1r, i2r, i3r, o_ref):
        acc = jnp.sum(u0r[...] * i0r[...], axis=1, keepdims=True)
        acc += jnp.sum(u1r[...] * i1r[...], axis=1, keepdims=True)
        acc += jnp.sum(u2r[...] * i2r[...], axis=1, keepdims=True)
        acc += jnp.sum(u3r[...] * i3r[...], axis=1, keepdims=True)
        o_ref[...] = acc

    out = pl.pallas_call(
        body,
        out_shape=jax.ShapeDtypeStruct((BATCH, 1), jnp.float32),
    )(u0, u1, u2, u3, i0, i1, i2, i3)
    return out.reshape(BATCH)


def kernel(edge_vals, user_embed, entity_embed,
           W_gc_0, b_gc_0, W_bi_0, b_bi_0,
           W_gc_1, b_gc_1, W_bi_1, b_bi_1,
           W_gc_2, b_gc_2, W_bi_2, b_bi_2,
           edge_index, users, pos_items):
    f32 = jnp.float32
    xcat = jnp.concatenate([user_embed, entity_embed], axis=0)
    npad = E_PAD - N_EDGES
    spread = (jnp.arange(npad, dtype=jnp.int32) * 16) % N_NODES
    dst2d = jnp.concatenate([edge_index[0], spread]).reshape(NWIN, EW)
    src2d = jnp.concatenate([edge_index[1], spread]).reshape(NWIN, EW)
    ev2d = jax.lax.bitcast_convert_type(
        jnp.concatenate([edge_vals, jnp.zeros((npad,), f32)]),
        jnp.int32).reshape(NWIN, EW)
    packed = jnp.stack([src2d, dst2d, ev2d], axis=1)
    z32 = jnp.zeros((N_NODES, 32), f32)
    z16 = jnp.zeros((N_NODES, 16), f32)

    xs0 = jnp.concatenate([xcat[:, :32], xcat[:, 32:]], axis=0)
    side0 = _spmm_sc(xs0, packed, z32, 32, feature_split=True, G=3)
    x1 = _dense_tc(xcat, side0, W_gc_0, b_gc_0, W_bi_0, b_bi_0,
                   concat_mode=True, pad_to=32)

    x1s = jnp.concatenate([x1[:, :16], x1[:, 16:]], axis=0)
    side1 = _spmm_sc(x1s, packed, z16, 16, feature_split=True, G=9)
    x2 = _dense_tc(x1, side1, W_gc_1, b_gc_1, W_bi_1, b_bi_1,
                   concat_mode=True, pad_to=16)

    side2 = _spmm_sc(x2, packed, z16, 16, feature_split=False, G=9)
    x3 = _dense_tc(x2, side2, W_gc_2, b_gc_2, W_bi_2, b_bi_2,
                   concat_mode=False, pad_to=16)

    parts = _gather_sc(xcat, x1, x2, x3, users, pos_items)
    return _dot_tc(parts)


# speedup vs baseline: 10.6354x; 1.0364x over previous
import functools

import jax
import jax.numpy as jnp
from jax import lax
from jax.experimental import pallas as pl
from jax.experimental.pallas import tpu as pltpu
from jax.experimental.pallas import tpu_sc as plsc

N_USERS = 10000
N_NODES = 50000
N_EDGES = 800000
EMB_DIM = 64
BATCH = 1024

EW = 128
NSUB = 16
NCORE = 2
NWIN = 6336
E_PAD = NWIN * EW


STRIPE = 3128
STRIPE_LAST = N_NODES - 15 * STRIPE


def _spmm_sc(x, packed, zeros, d, feature_split, G):
    n = N_NODES
    mesh = plsc.VectorSubcoreMesh(core_axis_name="c", subcore_axis_name="s")

    if feature_split:
        wps = NWIN // NSUB
    else:
        wps = NWIN // NCORE // NSUB
    npairs = wps // (2 * G)
    assert npairs * 2 * G == wps

    @functools.partial(
        pl.kernel,
        out_type=jax.ShapeDtypeStruct((2 * n, d), jnp.float32),
        mesh=mesh,
        scratch_types=[
            pltpu.VMEM((G, 3, EW), jnp.int32),
            pltpu.VMEM((G * EW, d), jnp.float32),
            pltpu.VMEM((G, 3, EW), jnp.int32),
            pltpu.VMEM((G * EW, d), jnp.float32),
            pltpu.VMEM_SHARED((n, d), jnp.float32),
            pltpu.SemaphoreType.DMA,
            pltpu.SemaphoreType.DMA,
            pltpu.SemaphoreType.DMA,
            pltpu.SemaphoreType.DMA,
        ],
        compiler_params=pltpu.CompilerParams(use_tc_tiling_on_sc=False,
                                             needs_layout_passes=False),
    )
    def spmm(x_hbm, pk_hbm, z_hbm, out_hbm,
             pkA, rowsA, pkB, rowsB,
             side, gsemA, ssemA, gsemB, ssemB):
        c = lax.axis_index("c")
        s = lax.axis_index("s")

        @pl.when(s < 15)
        def _():
            pltpu.sync_copy(z_hbm.at[pl.ds(s * STRIPE, STRIPE)],
                            side.at[pl.ds(s * STRIPE, STRIPE)])

        @pl.when(s == 15)
        def _():
            pltpu.sync_copy(z_hbm.at[pl.ds(15 * STRIPE, STRIPE_LAST)],
                            side.at[pl.ds(15 * STRIPE, STRIPE_LAST)])

        plsc.subcore_barrier()

        if feature_split:
            base_w = s * wps
        else:
            base_w = c * (NWIN // NCORE) + s * wps
        xoff = c * n

        def do_load(S, gi):
            pk, rows, gsem, _ = S
            w0 = base_w + gi * G
            pltpu.sync_copy(pk_hbm.at[pl.ds(w0, G)], pk)
            if feature_split:
                for j in range(G):
                    for u in range(EW // 16):
                        sl = pl.ds(u * 16, 16)
                        pk[j, 0, sl] = pk[j, 0, sl] + xoff
            for j in range(G):
                pltpu.async_copy(x_hbm.at[pk.at[j, 0]],
                                 rows.at[pl.ds(j * EW, EW)], gsem)

        def drain_gathers(S):
            pk, rows, gsem, _ = S
            for j in range(G):
                pltpu.make_async_copy(x_hbm.at[pk.at[j, 0]],
                                      rows.at[pl.ds(j * EW, EW)], gsem).wait()

        def do_scale(S):
            pk, rows = S[0], S[1]

            @plsc.parallel_loop(0, G * (EW // 16), unroll=2)
            def _(q):
                j = q // (EW // 16)
                cc = q % (EW // 16)
                ev16 = plsc.bitcast(pk[j, 2, pl.ds(cc * 16, 16)],
                                    jnp.float32)
                r0 = j * EW + cc * 16
                for l in range(16):
                    sv = ev16[l]
                    for jj in range(d // 16):
                        sl = pl.ds(jj * 16, 16)
                        rows[r0 + l, sl] = rows[r0 + l, sl] * sv

        def do_fire(S):
            pk, rows, _, ssem = S
            for j in range(G):
                pltpu.async_copy(rows.at[pl.ds(j * EW, EW)],
                                 side.at[pk.at[j, 1]], ssem, add=True)

        def drain_scatters(S):
            pk, rows, _, ssem = S
            for j in range(G):
                pltpu.make_async_copy(rows.at[pl.ds(j * EW, EW)],
                                      side.at[pk.at[j, 1]], ssem).wait()

        A = (pkA, rowsA, gsemA, ssemA)
        B = (pkB, rowsB, gsemB, ssemB)
        do_load(A, 0)

        @pl.loop(0, npairs)
        def _(p):
            drain_gathers(A)
            do_scale(A)

            @pl.when(p > 0)
            def _():
                drain_scatters(B)

            do_fire(A)
            do_load(B, 2 * p + 1)
            drain_gathers(B)
            do_scale(B)
            drain_scatters(A)
            do_fire(B)

            @pl.when(p < npairs - 1)
            def _():
                do_load(A, 2 * p + 2)

        drain_scatters(B)
        plsc.subcore_barrier()

        @pl.when(s < 15)
        def _():
            pltpu.sync_copy(side.at[pl.ds(s * STRIPE, STRIPE)],
                            out_hbm.at[pl.ds(c * n + s * STRIPE, STRIPE)])

        @pl.when(s == 15)
        def _():
            pltpu.sync_copy(side.at[pl.ds(15 * STRIPE, STRIPE_LAST)],
                            out_hbm.at[pl.ds(c * n + 15 * STRIPE, STRIPE_LAST)])

    return spmm(x, packed, zeros)


def _dense_tc(x, side2, Wg, bg, Wb, bb, concat_mode, pad_to):
    n, d_in = x.shape
    d_out = Wg.shape[1]
    R = 2000
    nblk = n // R

    def body(x_ref, s0_ref, s1_ref, wg_ref, bg_ref, wb_ref, bb_ref, o_ref):
        xb = x_ref[...]
        if concat_mode:
            side = jnp.concatenate([s0_ref[...], s1_ref[...]], axis=1)
        else:
            side = s0_ref[...] + s1_ref[...]
        a = jnp.dot(xb + side, wg_ref[...],
                    preferred_element_type=jnp.float32) + bg_ref[...]
        a = jnp.where(a >= 0, a, 0.01 * a)
        b = jnp.dot(xb * side, wb_ref[...],
                    preferred_element_type=jnp.float32) + bb_ref[...]
        b = jnp.where(b >= 0, b, 0.01 * b)
        y = a + b
        nrm = jnp.sqrt(jnp.sum(y * y, axis=1, keepdims=True))
        y = y / jnp.maximum(nrm, 1e-12)
        if pad_to > d_out:
            y = jnp.concatenate(
                [y, jnp.zeros((y.shape[0], pad_to - d_out), jnp.float32)],
                axis=1)
        o_ref[...] = y

    d_side = side2.shape[1]
    return pl.pallas_call(
        body,
        grid=(nblk,),
        in_specs=[
            pl.BlockSpec((R, d_in), lambda i: (i, 0)),
            pl.BlockSpec((R, d_side), lambda i: (i, 0)),
            pl.BlockSpec((R, d_side), lambda i: (i + nblk, 0)),
            pl.BlockSpec((d_in, d_out), lambda i: (0, 0)),
            pl.BlockSpec((1, d_out), lambda i: (0, 0)),
            pl.BlockSpec((d_in, d_out), lambda i: (0, 0)),
            pl.BlockSpec((1, d_out), lambda i: (0, 0)),
        ],
        out_specs=pl.BlockSpec((R, pad_to), lambda i: (i, 0)),
        out_shape=jax.ShapeDtypeStruct((n, pad_to), jnp.float32),
    )(x, side2, side2, Wg, bg, Wb, bb)


def _gather_sc(x0, x1, x2, x3, users, items):
    per_w = BATCH // (NCORE * NSUB)
    mesh = plsc.VectorSubcoreMesh(core_axis_name="c", subcore_axis_name="s")
    f32 = jnp.float32
    out_types = tuple(
        jax.ShapeDtypeStruct((BATCH, dd), f32) for dd in (64, 32, 16, 16)
    ) * 2

    @functools.partial(
        pl.kernel,
        out_type=out_types,
        mesh=mesh,
        scratch_types=[
            pltpu.VMEM((1, per_w), jnp.int32),
            pltpu.VMEM((per_w, 64), f32),
            pltpu.VMEM((per_w, 32), f32),
            pltpu.VMEM((per_w, 16), f32),
            pltpu.VMEM((per_w, 16), f32),
        ],
        compiler_params=pltpu.CompilerParams(use_tc_tiling_on_sc=False),
    )
    def gat(x0_hbm, x1_hbm, x2_hbm, x3_hbm, u_hbm, i_hbm,
            u0, u1, u2, u3, i0, i1, i2, i3,
            idx, r0, r1, r2, r3):
        c = lax.axis_index("c")
        s = lax.axis_index("s")
        wid = s * NCORE + c
        base = wid * per_w

        def do(ind_hbm, off, o0, o1, o2, o3):
            pltpu.sync_copy(ind_hbm.at[pl.ds(base, per_w)], idx.at[0])
            if off:
                for j in range(per_w // 16):
                    sl = pl.ds(j * 16, 16)
                    idx[0, sl] = idx[0, sl] + off
            pltpu.sync_copy(x0_hbm.at[idx.at[0]], r0)
            pltpu.sync_copy(r0, o0.at[pl.ds(base, per_w)])
            pltpu.sync_copy(x1_hbm.at[idx.at[0]], r1)
            pltpu.sync_copy(r1, o1.at[pl.ds(base, per_w)])
            pltpu.sync_copy(x2_hbm.at[idx.at[0]], r2)
            pltpu.sync_copy(r2, o2.at[pl.ds(base, per_w)])
            pltpu.sync_copy(x3_hbm.at[idx.at[0]], r3)
            pltpu.sync_copy(r3, o3.at[pl.ds(base, per_w)])

        do(u_hbm, 0, u0, u1, u2, u3)
        do(i_hbm, N_USERS, i0, i1, i2, i3)

    return gat(x0, x1, x2, x3, users, items)


def _dot_tc(parts):
    u0, u1, u2, u3, i0, i1, i2, i3 = parts

    def body(u0r, u1r, u2r, u3r, i0r, i1r, i2r, i3r, o_ref):
        acc = jnp.sum(u0r[...] * i0r[...], axis=1, keepdims=True)
        acc += jnp.sum(u1r[...] * i1r[...], axis=1, keepdims=True)
        acc += jnp.sum(u2r[...] * i2r[...], axis=1, keepdims=True)
        acc += jnp.sum(u3r[...] * i3r[...], axis=1, keepdims=True)
        o_ref[...] = acc

    out = pl.pallas_call(
        body,
        out_shape=jax.ShapeDtypeStruct((BATCH, 1), jnp.float32),
    )(u0, u1, u2, u3, i0, i1, i2, i3)
    return out.reshape(BATCH)


def kernel(edge_vals, user_embed, entity_embed,
           W_gc_0, b_gc_0, W_bi_0, b_bi_0,
           W_gc_1, b_gc_1, W_bi_1, b_bi_1,
           W_gc_2, b_gc_2, W_bi_2, b_bi_2,
           edge_index, users, pos_items):
    f32 = jnp.float32
    xcat = jnp.concatenate([user_embed, entity_embed], axis=0)
    npad = E_PAD - N_EDGES
    spread = (jnp.arange(npad, dtype=jnp.int32) * 16) % N_NODES
    dst2d = jnp.concatenate([edge_index[0], spread]).reshape(NWIN, EW)
    src2d = jnp.concatenate([edge_index[1], spread]).reshape(NWIN, EW)
    ev2d = jax.lax.bitcast_convert_type(
        jnp.concatenate([edge_vals, jnp.zeros((npad,), f32)]),
        jnp.int32).reshape(NWIN, EW)
    packed = jnp.stack([src2d, dst2d, ev2d], axis=1)
    z32 = jnp.zeros((N_NODES, 32), f32)
    z16 = jnp.zeros((N_NODES, 16), f32)

    xs0 = jnp.concatenate([xcat[:, :32], xcat[:, 32:]], axis=0)
    side0 = _spmm_sc(xs0, packed, z32, 32, feature_split=True, G=3)
    x1 = _dense_tc(xcat, side0, W_gc_0, b_gc_0, W_bi_0, b_bi_0,
                   concat_mode=True, pad_to=32)

    x1s = jnp.concatenate([x1[:, :16], x1[:, 16:]], axis=0)
    side1 = _spmm_sc(x1s, packed, z16, 16, feature_split=True, G=9)
    x2 = _dense_tc(x1, side1, W_gc_1, b_gc_1, W_bi_1, b_bi_1,
                   concat_mode=True, pad_to=16)

    side2 = _spmm_sc(x2, packed, z16, 16, feature_split=False, G=9)
    x3 = _dense_tc(x2, side2, W_gc_2, b_gc_2, W_bi_2, b_bi_2,
                   concat_mode=False, pad_to=16)

    parts = _gather_sc(xcat, x1, x2, x3, users, pos_items)
    return _dot_tc(parts)


# speedup vs baseline: 11.4865x; 1.0800x over previous
import functools

import jax
import jax.numpy as jnp
from jax import lax
from jax.experimental import pallas as pl
from jax.experimental.pallas import tpu as pltpu
from jax.experimental.pallas import tpu_sc as plsc

N_USERS = 10000
N_NODES = 50000
N_EDGES = 800000
EMB_DIM = 64
BATCH = 1024

EW = 128
NSUB = 16
NCORE = 2
NWIN = 6336
E_PAD = NWIN * EW


STRIPE = 3128
STRIPE_LAST = N_NODES - 15 * STRIPE


def _spmm_sc(xa, xb, packed, zeros, d, feature_split, G):
    n = N_NODES
    mesh = plsc.VectorSubcoreMesh(core_axis_name="c", subcore_axis_name="s")

    if feature_split:
        wps = NWIN // NSUB
    else:
        wps = NWIN // NCORE // NSUB
    npairs = wps // (2 * G)
    assert npairs * 2 * G == wps

    @functools.partial(
        pl.kernel,
        out_type=jax.ShapeDtypeStruct((2 * n, d), jnp.float32),
        mesh=mesh,
        scratch_types=[
            pltpu.VMEM((G, 3, EW), jnp.int32),
            pltpu.VMEM((G * EW, d), jnp.float32),
            pltpu.VMEM((G, 3, EW), jnp.int32),
            pltpu.VMEM((G * EW, d), jnp.float32),
            pltpu.VMEM_SHARED((n, d), jnp.float32),
            pltpu.SemaphoreType.DMA,
            pltpu.SemaphoreType.DMA,
            pltpu.SemaphoreType.DMA,
            pltpu.SemaphoreType.DMA,
        ],
        compiler_params=pltpu.CompilerParams(use_tc_tiling_on_sc=False,
                                             needs_layout_passes=False),
    )
    def spmm(xa_hbm, xb_hbm, pk_hbm, z_hbm, out_hbm,
             pkA, rowsA, pkB, rowsB,
             side, gsemA, ssemA, gsemB, ssemB):
        c = lax.axis_index("c")
        s = lax.axis_index("s")

        @pl.when(s < 15)
        def _():
            pltpu.sync_copy(z_hbm.at[pl.ds(s * STRIPE, STRIPE)],
                            side.at[pl.ds(s * STRIPE, STRIPE)])

        @pl.when(s == 15)
        def _():
            pltpu.sync_copy(z_hbm.at[pl.ds(15 * STRIPE, STRIPE_LAST)],
                            side.at[pl.ds(15 * STRIPE, STRIPE_LAST)])

        plsc.subcore_barrier()

        if feature_split:
            base_w = s * wps
        else:
            base_w = c * (NWIN // NCORE) + s * wps

        def do_load(S, gi):
            pk, rows, gsem, _ = S
            w0 = base_w + gi * G
            pltpu.sync_copy(pk_hbm.at[pl.ds(w0, G)], pk)

            @pl.when(c == 0)
            def _():
                for j in range(G):
                    pltpu.async_copy(xa_hbm.at[pk.at[j, 0]],
                                     rows.at[pl.ds(j * EW, EW)], gsem)

            @pl.when(c == 1)
            def _():
                for j in range(G):
                    pltpu.async_copy(xb_hbm.at[pk.at[j, 0]],
                                     rows.at[pl.ds(j * EW, EW)], gsem)

        def drain_gathers(S):
            pk, rows, gsem, _ = S

            @pl.when(c == 0)
            def _():
                for j in range(G):
                    pltpu.make_async_copy(
                        xa_hbm.at[pk.at[j, 0]],
                        rows.at[pl.ds(j * EW, EW)], gsem).wait()

            @pl.when(c == 1)
            def _():
                for j in range(G):
                    pltpu.make_async_copy(
                        xb_hbm.at[pk.at[j, 0]],
                        rows.at[pl.ds(j * EW, EW)], gsem).wait()

        def do_scale(S):
            pk, rows = S[0], S[1]

            @plsc.parallel_loop(0, G * (EW // 16), unroll=2)
            def _(q):
                j = q // (EW // 16)
                cc = q % (EW // 16)
                ev16 = plsc.bitcast(pk[j, 2, pl.ds(cc * 16, 16)],
                                    jnp.float32)
                r0 = j * EW + cc * 16
                for l in range(16):
                    sv = ev16[l]
                    for jj in range(d // 16):
                        sl = pl.ds(jj * 16, 16)
                        rows[r0 + l, sl] = rows[r0 + l, sl] * sv

        def do_fire(S):
            pk, rows, _, ssem = S
            for j in range(G):
                pltpu.async_copy(rows.at[pl.ds(j * EW, EW)],
                                 side.at[pk.at[j, 1]], ssem, add=True)

        def drain_scatters(S):
            pk, rows, _, ssem = S
            for j in range(G):
                pltpu.make_async_copy(rows.at[pl.ds(j * EW, EW)],
                                      side.at[pk.at[j, 1]], ssem).wait()

        A = (pkA, rowsA, gsemA, ssemA)
        B = (pkB, rowsB, gsemB, ssemB)
        do_load(A, 0)

        @pl.loop(0, npairs)
        def _(p):
            drain_gathers(A)
            do_scale(A)

            @pl.when(p > 0)
            def _():
                drain_scatters(B)

            do_fire(A)
            do_load(B, 2 * p + 1)
            drain_gathers(B)
            do_scale(B)
            drain_scatters(A)
            do_fire(B)

            @pl.when(p < npairs - 1)
            def _():
                do_load(A, 2 * p + 2)

        drain_scatters(B)
        plsc.subcore_barrier()

        @pl.when(s < 15)
        def _():
            pltpu.sync_copy(side.at[pl.ds(s * STRIPE, STRIPE)],
                            out_hbm.at[pl.ds(c * n + s * STRIPE, STRIPE)])

        @pl.when(s == 15)
        def _():
            pltpu.sync_copy(side.at[pl.ds(15 * STRIPE, STRIPE_LAST)],
                            out_hbm.at[pl.ds(c * n + 15 * STRIPE, STRIPE_LAST)])

    return spmm(xa, xb, packed, zeros)


def _prep_x0(user_embed, entity_embed):
    R = 2000
    nblk = N_NODES // R
    nu_blk = N_USERS // R

    def body(u_ref, e_ref, o_ref, lo_ref, hi_ref):
        i = pl.program_id(0)
        v = jnp.where(i < nu_blk, u_ref[...], e_ref[...])
        o_ref[...] = v
        lo_ref[...] = v[:, :32]
        hi_ref[...] = v[:, 32:]

    f32 = jnp.float32
    return pl.pallas_call(
        body,
        grid=(nblk,),
        in_specs=[
            pl.BlockSpec((R, EMB_DIM),
                         lambda i: (jnp.minimum(i, nu_blk - 1), 0)),
            pl.BlockSpec((R, EMB_DIM),
                         lambda i: (jnp.maximum(i - nu_blk, 0), 0)),
        ],
        out_specs=[
            pl.BlockSpec((R, EMB_DIM), lambda i: (i, 0)),
            pl.BlockSpec((R, 32), lambda i: (i, 0)),
            pl.BlockSpec((R, 32), lambda i: (i, 0)),
        ],
        out_shape=[
            jax.ShapeDtypeStruct((N_NODES, EMB_DIM), f32),
            jax.ShapeDtypeStruct((N_NODES, 32), f32),
            jax.ShapeDtypeStruct((N_NODES, 32), f32),
        ],
    )(user_embed, entity_embed)


def _dense_tc(x, side2, Wg, bg, Wb, bb, concat_mode, pad_to,
              emit_halves=False):
    n, d_in = x.shape
    d_out = Wg.shape[1]
    R = 5000
    nblk = n // R

    def body(x_ref, s0_ref, s1_ref, wg_ref, bg_ref, wb_ref, bb_ref,
             o_ref, *half_refs):
        xb = x_ref[...]
        if concat_mode:
            side = jnp.concatenate([s0_ref[...], s1_ref[...]], axis=1)
        else:
            side = s0_ref[...] + s1_ref[...]
        a = jnp.dot(xb + side, wg_ref[...],
                    preferred_element_type=jnp.float32) + bg_ref[...]
        a = jnp.where(a >= 0, a, 0.01 * a)
        b = jnp.dot(xb * side, wb_ref[...],
                    preferred_element_type=jnp.float32) + bb_ref[...]
        b = jnp.where(b >= 0, b, 0.01 * b)
        y = a + b
        nrm = jnp.sqrt(jnp.sum(y * y, axis=1, keepdims=True))
        y = y / jnp.maximum(nrm, 1e-12)
        if emit_halves:
            half_refs[0][...] = y[:, :d_out // 2]
            half_refs[1][...] = y[:, d_out // 2:]
        if pad_to > d_out:
            y = jnp.concatenate(
                [y, jnp.zeros((y.shape[0], pad_to - d_out), jnp.float32)],
                axis=1)
        o_ref[...] = y

    d_side = side2.shape[1]
    out_specs = [pl.BlockSpec((R, pad_to), lambda i: (i, 0))]
    out_shape = [jax.ShapeDtypeStruct((n, pad_to), jnp.float32)]
    if emit_halves:
        for _ in range(2):
            out_specs.append(pl.BlockSpec((R, d_out // 2), lambda i: (i, 0)))
            out_shape.append(jax.ShapeDtypeStruct((n, d_out // 2),
                                                  jnp.float32))
    res = pl.pallas_call(
        body,
        grid=(nblk,),
        in_specs=[
            pl.BlockSpec((R, d_in), lambda i: (i, 0)),
            pl.BlockSpec((R, d_side), lambda i: (i, 0)),
            pl.BlockSpec((R, d_side), lambda i: (i + nblk, 0)),
            pl.BlockSpec((d_in, d_out), lambda i: (0, 0)),
            pl.BlockSpec((1, d_out), lambda i: (0, 0)),
            pl.BlockSpec((d_in, d_out), lambda i: (0, 0)),
            pl.BlockSpec((1, d_out), lambda i: (0, 0)),
        ],
        out_specs=out_specs if emit_halves else out_specs[0],
        out_shape=out_shape if emit_halves else out_shape[0],
    )(x, side2, side2, Wg, bg, Wb, bb)
    return res


def _gather_sc(x0, x1, x2, x3, users, items):
    per_w = BATCH // (NCORE * NSUB)
    mesh = plsc.VectorSubcoreMesh(core_axis_name="c", subcore_axis_name="s")
    f32 = jnp.float32
    out_types = tuple(
        jax.ShapeDtypeStruct((BATCH, dd), f32) for dd in (64, 32, 16, 16)
    ) * 2

    @functools.partial(
        pl.kernel,
        out_type=out_types,
        mesh=mesh,
        scratch_types=[
            pltpu.VMEM((1, per_w), jnp.int32),
            pltpu.VMEM((per_w, 64), f32),
            pltpu.VMEM((per_w, 32), f32),
            pltpu.VMEM((per_w, 16), f32),
            pltpu.VMEM((per_w, 16), f32),
        ],
        compiler_params=pltpu.CompilerParams(use_tc_tiling_on_sc=False),
    )
    def gat(x0_hbm, x1_hbm, x2_hbm, x3_hbm, u_hbm, i_hbm,
            u0, u1, u2, u3, i0, i1, i2, i3,
            idx, r0, r1, r2, r3):
        c = lax.axis_index("c")
        s = lax.axis_index("s")
        wid = s * NCORE + c
        base = wid * per_w

        def do(ind_hbm, off, o0, o1, o2, o3):
            pltpu.sync_copy(ind_hbm.at[pl.ds(base, per_w)], idx.at[0])
            if off:
                for j in range(per_w // 16):
                    sl = pl.ds(j * 16, 16)
                    idx[0, sl] = idx[0, sl] + off
            pltpu.sync_copy(x0_hbm.at[idx.at[0]], r0)
            pltpu.sync_copy(r0, o0.at[pl.ds(base, per_w)])
            pltpu.sync_copy(x1_hbm.at[idx.at[0]], r1)
            pltpu.sync_copy(r1, o1.at[pl.ds(base, per_w)])
            pltpu.sync_copy(x2_hbm.at[idx.at[0]], r2)
            pltpu.sync_copy(r2, o2.at[pl.ds(base, per_w)])
            pltpu.sync_copy(x3_hbm.at[idx.at[0]], r3)
            pltpu.sync_copy(r3, o3.at[pl.ds(base, per_w)])

        do(u_hbm, 0, u0, u1, u2, u3)
        do(i_hbm, N_USERS, i0, i1, i2, i3)

    return gat(x0, x1, x2, x3, users, items)


def _dot_tc(parts):
    u0, u1, u2, u3, i0, i1, i2, i3 = parts

    def body(u0r, u1r, u2r, u3r, i0r, i1r, i2r, i3r, o_ref):
        acc = jnp.sum(u0r[...] * i0r[...], axis=1, keepdims=True)
        acc += jnp.sum(u1r[...] * i1r[...], axis=1, keepdims=True)
        acc += jnp.sum(u2r[...] * i2r[...], axis=1, keepdims=True)
        acc += jnp.sum(u3r[...] * i3r[...], axis=1, keepdims=True)
        o_ref[...] = acc

    out = pl.pallas_call(
        body,
        out_shape=jax.ShapeDtypeStruct((BATCH, 1), jnp.float32),
    )(u0, u1, u2, u3, i0, i1, i2, i3)
    return out.reshape(BATCH)


def kernel(edge_vals, user_embed, entity_embed,
           W_gc_0, b_gc_0, W_bi_0, b_bi_0,
           W_gc_1, b_gc_1, W_bi_1, b_bi_1,
           W_gc_2, b_gc_2, W_bi_2, b_bi_2,
           edge_index, users, pos_items):
    f32 = jnp.float32
    npad = E_PAD - N_EDGES
    spread = (jnp.arange(npad, dtype=jnp.int32) * 16) % N_NODES
    dst2d = jnp.concatenate([edge_index[0], spread]).reshape(NWIN, EW)
    src2d = jnp.concatenate([edge_index[1], spread]).reshape(NWIN, EW)
    ev2d = jax.lax.bitcast_convert_type(
        jnp.concatenate([edge_vals, jnp.zeros((npad,), f32)]),
        jnp.int32).reshape(NWIN, EW)
    packed = jnp.stack([src2d, dst2d, ev2d], axis=1)
    z32 = jnp.zeros((N_NODES, 32), f32)
    z16 = jnp.zeros((N_NODES, 16), f32)

    xcat, x0_lo, x0_hi = _prep_x0(user_embed, entity_embed)

    side0 = _spmm_sc(x0_lo, x0_hi, packed, z32, 32, feature_split=True, G=3)
    x1, x1_lo, x1_hi = _dense_tc(xcat, side0, W_gc_0, b_gc_0, W_bi_0, b_bi_0,
                                 concat_mode=True, pad_to=32,
                                 emit_halves=True)

    side1 = _spmm_sc(x1_lo, x1_hi, packed, z16, 16, feature_split=True, G=9)
    x2 = _dense_tc(x1, side1, W_gc_1, b_gc_1, W_bi_1, b_bi_1,
                   concat_mode=True, pad_to=16)

    side2 = _spmm_sc(x2, x2, packed, z16, 16, feature_split=False, G=9)
    x3 = _dense_tc(x2, side2, W_gc_2, b_gc_2, W_bi_2, b_bi_2,
                   concat_mode=False, pad_to=16)

    parts = _gather_sc(xcat, x1, x2, x3, users, pos_items)
    return _dot_tc(parts)
